# baseline (speedup 1.0000x reference)
import functools

import jax
import jax.numpy as jnp
from jax import lax
from jax.experimental import pallas as pl
from jax.experimental.pallas import tpu as pltpu
from jax.experimental.pallas import tpu_sc as plsc

VOCAB = 1000000
DIM = 64
BATCH = 16384

_info = plsc.get_sparse_core_info()
NC, NS, L = _info.num_cores, _info.num_subcores, _info.num_lanes
NW = NC * NS
BPW = BATCH // NW


def _sc_body(w_i_hbm, w_j_hbm, w_emb_hbm, c_emb_hbm, w_bias_hbm, c_bias_hbm,
             partials_hbm, bias_hbm,
             idx_i_v, idx_j_v, rows_i_v, rows_j_v, bi_v, bj_v, acc_v,
             sem_i, sem_j, sem_bi, sem_bj):
    wid = lax.axis_index("s") * NC + lax.axis_index("c")
    base = wid * BPW

    pltpu.sync_copy(w_i_hbm.at[pl.ds(base, BPW)], idx_i_v)
    pltpu.sync_copy(w_j_hbm.at[pl.ds(base, BPW)], idx_j_v)

    cp_i = pltpu.async_copy(w_emb_hbm.at[idx_i_v], rows_i_v, sem_i)
    cp_j = pltpu.async_copy(c_emb_hbm.at[idx_j_v], rows_j_v, sem_j)
    cp_bi = pltpu.async_copy(w_bias_hbm.at[idx_i_v], bi_v, sem_bi)
    cp_bj = pltpu.async_copy(c_bias_hbm.at[idx_j_v], bj_v, sem_bj)

    cp_bi.wait()
    cp_bj.wait()

    def bias_step(k, _):
        s = pl.ds(k * L, L)
        bi_v[s] = bi_v[s] + bj_v[s]
        return 0

    lax.fori_loop(0, BPW // L, bias_step, 0, unroll=4)
    pltpu.sync_copy(bi_v, bias_hbm.at[pl.ds(base, BPW)])

    cp_i.wait()
    cp_j.wait()

    def dot_step(r, accs):
        a0, a1, a2, a3 = accs
        a0 = a0 + rows_i_v[r, pl.ds(0 * L, L)] * rows_j_v[r, pl.ds(0 * L, L)]
        a1 = a1 + rows_i_v[r, pl.ds(1 * L, L)] * rows_j_v[r, pl.ds(1 * L, L)]
        a2 = a2 + rows_i_v[r, pl.ds(2 * L, L)] * rows_j_v[r, pl.ds(2 * L, L)]
        a3 = a3 + rows_i_v[r, pl.ds(3 * L, L)] * rows_j_v[r, pl.ds(3 * L, L)]
        return (a0, a1, a2, a3)

    zero = jnp.zeros((L,), jnp.float32)
    a0, a1, a2, a3 = lax.fori_loop(0, BPW, dot_step, (zero, zero, zero, zero),
                                   unroll=4)
    acc_v[...] = (a0 + a1) + (a2 + a3)
    pltpu.sync_copy(acc_v, partials_hbm.at[wid])


def _sc_gather(w_i, w_j, w_emb, c_emb, w_bias, c_bias):
    mesh = plsc.VectorSubcoreMesh(core_axis_name="c", subcore_axis_name="s")
    f = pl.kernel(
        _sc_body,
        out_type=(
            jax.ShapeDtypeStruct((NW, L), jnp.float32),
            jax.ShapeDtypeStruct((BATCH,), jnp.float32),
        ),
        mesh=mesh,
        compiler_params=pltpu.CompilerParams(use_tc_tiling_on_sc=False),
        scratch_types=[
            pltpu.VMEM((BPW,), jnp.int32),
            pltpu.VMEM((BPW,), jnp.int32),
            pltpu.VMEM((BPW, DIM), jnp.float32),
            pltpu.VMEM((BPW, DIM), jnp.float32),
            pltpu.VMEM((BPW,), jnp.float32),
            pltpu.VMEM((BPW,), jnp.float32),
            pltpu.VMEM((L,), jnp.float32),
            pltpu.SemaphoreType.DMA,
            pltpu.SemaphoreType.DMA,
            pltpu.SemaphoreType.DMA,
            pltpu.SemaphoreType.DMA,
        ],
    )
    return f(w_i, w_j, w_emb, c_emb, w_bias, c_bias)


def _tc_body(partials_ref, bias_ref, x_ref, loss_ref):
    x = jnp.sum(partials_ref[...])
    b = bias_ref[...]
    y_true = jnp.abs(b) + 1e-6
    t = jnp.abs(x) / 100.0 + jnp.zeros_like(b)
    weight = jnp.exp(0.75 * jnp.log(t))
    loss_ref[...] = weight * jnp.square(x - jnp.log(y_true))
    x_ref[...] = jnp.broadcast_to(x, (1, 1))


def _tc_loss(partials, bias2d):
    return pl.pallas_call(
        _tc_body,
        out_shape=(
            jax.ShapeDtypeStruct((1, 1), jnp.float32),
            jax.ShapeDtypeStruct(bias2d.shape, jnp.float32),
        ),
    )(partials, bias2d)


def kernel(w_i, w_j, w_emb, c_emb, w_bias, c_bias):
    w_i = w_i.astype(jnp.int32)
    w_j = w_j.astype(jnp.int32)
    partials, bias = _sc_gather(w_i, w_j, w_emb, c_emb, w_bias, c_bias)
    x, loss = _tc_loss(partials, bias.reshape(128, 128))
    return (x.reshape(()), loss.reshape(BATCH))


# speedup vs baseline: 1.2097x; 1.2097x over previous
import jax
import jax.numpy as jnp
from jax import lax
from jax.experimental import pallas as pl
from jax.experimental.pallas import tpu as pltpu
from jax.experimental.pallas import tpu_sc as plsc

VOCAB = 1000000
DIM = 64
BATCH = 16384

_info = plsc.get_sparse_core_info()
NC, NS, L = _info.num_cores, _info.num_subcores, _info.num_lanes
NW = NC * NS
BPW = BATCH // NW
CHUNK = 128


def _dot_body(w_i_hbm, w_j_hbm, wp_hbm, cp_hbm, partials_hbm,
              idx_i_v, idx_j_v, row_i_v, row_j_v, rows_i_v, rows_j_v,
              acc_v, sem_i, sem_j):
    wid = lax.axis_index("s") * NC + lax.axis_index("c")
    base = wid * BPW

    pltpu.sync_copy(w_i_hbm.at[pl.ds(base, BPW)], idx_i_v)
    pltpu.sync_copy(w_j_hbm.at[pl.ds(base, BPW)], idx_j_v)

    def to_rows(k, _):
        s = pl.ds(k * L, L)
        iv = idx_i_v[s]
        jv = idx_j_v[s]
        row_i_v[s] = ((iv >> 10) << 9) | (iv & 511)
        row_j_v[s] = ((jv >> 10) << 9) | (jv & 511)
        return 0

    lax.fori_loop(0, BPW // L, to_rows, 0, unroll=4)

    zero = jnp.zeros((L,), jnp.float32)
    accs = (zero, zero, zero, zero)
    for chunk in range(BPW // CHUNK):
        cb = chunk * CHUNK
        cp_i = pltpu.async_copy(
            wp_hbm.at[row_i_v.at[pl.ds(cb, CHUNK)]], rows_i_v, sem_i)
        cp_j = pltpu.async_copy(
            cp_hbm.at[row_j_v.at[pl.ds(cb, CHUNK)]], rows_j_v, sem_j)
        cp_i.wait()
        cp_j.wait()

        def dot_group(g, accs):
            a0, a1, a2, a3 = accs
            iv = idx_i_v[pl.ds(cb + g * L, L)]
            jv = idx_j_v[pl.ds(cb + g * L, L)]
            for t in range(L):
                k = g * L + t
                oi = ((iv[t] >> 9) & 1) * DIM
                oj = ((jv[t] >> 9) & 1) * DIM
                a0 = a0 + rows_i_v[k, pl.ds(oi, L)] * rows_j_v[k, pl.ds(oj, L)]
                a1 = a1 + (rows_i_v[k, pl.ds(oi + L, L)]
                           * rows_j_v[k, pl.ds(oj + L, L)])
                a2 = a2 + (rows_i_v[k, pl.ds(oi + 2 * L, L)]
                           * rows_j_v[k, pl.ds(oj + 2 * L, L)])
                a3 = a3 + (rows_i_v[k, pl.ds(oi + 3 * L, L)]
                           * rows_j_v[k, pl.ds(oj + 3 * L, L)])
            return (a0, a1, a2, a3)

        accs = lax.fori_loop(0, CHUNK // L, dot_group, accs)

    a0, a1, a2, a3 = accs
    acc_v[pl.ds(0, L)] = a0
    acc_v[pl.ds(L, L)] = a1
    acc_v[pl.ds(2 * L, L)] = a2
    acc_v[pl.ds(3 * L, L)] = a3
    for z in range(4, 8):
        acc_v[pl.ds(z * L, L)] = zero
    pltpu.sync_copy(acc_v, partials_hbm.at[pl.ds(wid * 128, 128)])


def _sc_dot(w_i, w_j, wp, cp):
    mesh = plsc.VectorSubcoreMesh(core_axis_name="c", subcore_axis_name="s")
    f = pl.kernel(
        _dot_body,
        out_type=jax.ShapeDtypeStruct((NW * 128,), jnp.float32),
        mesh=mesh,
        scratch_types=[
            pltpu.VMEM((BPW,), jnp.int32),
            pltpu.VMEM((BPW,), jnp.int32),
            pltpu.VMEM((BPW,), jnp.int32),
            pltpu.VMEM((BPW,), jnp.int32),
            pltpu.VMEM((CHUNK, 2 * DIM), jnp.float32),
            pltpu.VMEM((CHUNK, 2 * DIM), jnp.float32),
            pltpu.VMEM((128,), jnp.float32),
            pltpu.SemaphoreType.DMA,
            pltpu.SemaphoreType.DMA,
        ],
    )
    return f(w_i, w_j, wp, cp)


def _bias_body(w_i_hbm, w_j_hbm, w_bias_hbm, c_bias_hbm, bias_hbm,
               idx_i_v, idx_j_v, bi_v, bj_v, sem_bi, sem_bj):
    wid = lax.axis_index("s") * NC + lax.axis_index("c")
    base = wid * BPW

    pltpu.sync_copy(w_i_hbm.at[pl.ds(base, BPW)], idx_i_v)
    pltpu.sync_copy(w_j_hbm.at[pl.ds(base, BPW)], idx_j_v)

    cp_bi = pltpu.async_copy(w_bias_hbm.at[idx_i_v], bi_v, sem_bi)
    cp_bj = pltpu.async_copy(c_bias_hbm.at[idx_j_v], bj_v, sem_bj)
    cp_bi.wait()
    cp_bj.wait()

    def bias_step(k, _):
        s = pl.ds(k * L, L)
        bi_v[s] = bi_v[s] + bj_v[s]
        return 0

    lax.fori_loop(0, BPW // L, bias_step, 0, unroll=4)
    pltpu.sync_copy(bi_v, bias_hbm.at[pl.ds(base, BPW)])


def _sc_bias(w_i, w_j, w_bias, c_bias):
    mesh = plsc.VectorSubcoreMesh(core_axis_name="c", subcore_axis_name="s")
    f = pl.kernel(
        _bias_body,
        out_type=jax.ShapeDtypeStruct((BATCH,), jnp.float32),
        mesh=mesh,
        compiler_params=pltpu.CompilerParams(use_tc_tiling_on_sc=False),
        scratch_types=[
            pltpu.VMEM((BPW,), jnp.int32),
            pltpu.VMEM((BPW,), jnp.int32),
            pltpu.VMEM((BPW,), jnp.float32),
            pltpu.VMEM((BPW,), jnp.float32),
            pltpu.SemaphoreType.DMA,
            pltpu.SemaphoreType.DMA,
        ],
    )
    return f(w_i, w_j, w_bias, c_bias)


_PACK_W = 1024
_PACK_GRID = (VOCAB + _PACK_W - 1) // _PACK_W
_PACK_ROWS = _PACK_GRID * 512


def _pack_body(wt_ref, ct_ref, ow_ref, oc_ref):
    w = wt_ref[...]
    ow_ref[...] = jnp.concatenate(
        [w[:, :512].T, w[:, 512:].T], axis=1)
    c = ct_ref[...]
    oc_ref[...] = jnp.concatenate(
        [c[:, :512].T, c[:, 512:].T], axis=1)


def _tc_pack(wt, ct):
    return pl.pallas_call(
        _pack_body,
        grid=(_PACK_GRID,),
        in_specs=[
            pl.BlockSpec((DIM, _PACK_W), lambda g: (0, g)),
            pl.BlockSpec((DIM, _PACK_W), lambda g: (0, g)),
        ],
        out_specs=[
            pl.BlockSpec((512, 128), lambda g: (g, 0)),
            pl.BlockSpec((512, 128), lambda g: (g, 0)),
        ],
        out_shape=(
            jax.ShapeDtypeStruct((_PACK_ROWS, 128), jnp.float32),
            jax.ShapeDtypeStruct((_PACK_ROWS, 128), jnp.float32),
        ),
    )(wt, ct)


def _tc_body(partials_ref, bias_ref, x_ref, loss_ref):
    x = jnp.sum(partials_ref[...])
    b = bias_ref[...]
    y_true = jnp.abs(b) + 1e-6
    t = jnp.abs(x) / 100.0 + jnp.zeros_like(b)
    weight = jnp.exp(0.75 * jnp.log(t))
    loss_ref[...] = weight * jnp.square(x - jnp.log(y_true))
    x_ref[...] = jnp.broadcast_to(x, (1, 1))


def _tc_loss(partials, bias2d):
    return pl.pallas_call(
        _tc_body,
        out_shape=(
            jax.ShapeDtypeStruct((1, 1), jnp.float32),
            jax.ShapeDtypeStruct(bias2d.shape, jnp.float32),
        ),
    )(partials, bias2d)


def kernel(w_i, w_j, w_emb, c_emb, w_bias, c_bias):
    w_i = w_i.astype(jnp.int32)
    w_j = w_j.astype(jnp.int32)
    wp, cp = _tc_pack(w_emb.T, c_emb.T)
    partials = _sc_dot(w_i, w_j, wp, cp)
    bias = _sc_bias(w_i, w_j, w_bias, c_bias)
    x, loss = _tc_loss(partials.reshape(NW, 128), bias.reshape(128, 128))
    return (x.reshape(()), loss.reshape(BATCH))


# speedup vs baseline: 2.0637x; 1.7060x over previous
import jax
import jax.numpy as jnp
from jax import lax
from jax.experimental import pallas as pl
from jax.experimental.pallas import tpu as pltpu
from jax.experimental.pallas import tpu_sc as plsc

VOCAB = 1000000
DIM = 64
BATCH = 16384

_info = plsc.get_sparse_core_info()
NC, NS, L = _info.num_cores, _info.num_subcores, _info.num_lanes
NW = NC * NS
BPW = BATCH // NW
CHUNK = 128


def _dot_body(w_i_hbm, w_j_hbm, wp_hbm, cp_hbm, partials_hbm,
              idx_i_v, idx_j_v, row_i_v, row_j_v, rows_i_v, rows_j_v,
              acc_v, sem_i, sem_j):
    wid = lax.axis_index("s") * NC + lax.axis_index("c")
    base = wid * BPW

    pltpu.sync_copy(w_i_hbm.at[pl.ds(base, BPW)], idx_i_v)
    pltpu.sync_copy(w_j_hbm.at[pl.ds(base, BPW)], idx_j_v)

    def to_rows(k, _):
        s = pl.ds(k * L, L)
        iv = idx_i_v[s]
        jv = idx_j_v[s]
        row_i_v[s] = ((iv >> 12) << 11) | (iv & 2047)
        row_j_v[s] = ((jv >> 12) << 11) | (jv & 2047)
        return 0

    lax.fori_loop(0, BPW // L, to_rows, 0, unroll=4)

    zero = jnp.zeros((L,), jnp.float32)
    accs = (zero, zero, zero, zero)
    for chunk in range(BPW // CHUNK):
        cb = chunk * CHUNK
        cp_i = pltpu.async_copy(
            wp_hbm.at[row_i_v.at[pl.ds(cb, CHUNK)]], rows_i_v, sem_i)
        cp_j = pltpu.async_copy(
            cp_hbm.at[row_j_v.at[pl.ds(cb, CHUNK)]], rows_j_v, sem_j)
        cp_i.wait()
        cp_j.wait()

        def dot_group(g, accs):
            a0, a1, a2, a3 = accs
            iv = idx_i_v[pl.ds(cb + g * L, L)]
            jv = idx_j_v[pl.ds(cb + g * L, L)]
            for t in range(L):
                k = g * L + t
                oi = ((iv[t] >> 11) & 1) * DIM
                oj = ((jv[t] >> 11) & 1) * DIM
                a0 = a0 + rows_i_v[k, pl.ds(oi, L)] * rows_j_v[k, pl.ds(oj, L)]
                a1 = a1 + (rows_i_v[k, pl.ds(oi + L, L)]
                           * rows_j_v[k, pl.ds(oj + L, L)])
                a2 = a2 + (rows_i_v[k, pl.ds(oi + 2 * L, L)]
                           * rows_j_v[k, pl.ds(oj + 2 * L, L)])
                a3 = a3 + (rows_i_v[k, pl.ds(oi + 3 * L, L)]
                           * rows_j_v[k, pl.ds(oj + 3 * L, L)])
            return (a0, a1, a2, a3)

        accs = lax.fori_loop(0, CHUNK // L, dot_group, accs)

    a0, a1, a2, a3 = accs
    acc_v[pl.ds(0, L)] = a0
    acc_v[pl.ds(L, L)] = a1
    acc_v[pl.ds(2 * L, L)] = a2
    acc_v[pl.ds(3 * L, L)] = a3
    for z in range(4, 8):
        acc_v[pl.ds(z * L, L)] = zero
    pltpu.sync_copy(acc_v, partials_hbm.at[pl.ds(wid * 128, 128)])


def _sc_dot(w_i, w_j, wp, cp):
    mesh = plsc.VectorSubcoreMesh(core_axis_name="c", subcore_axis_name="s")
    f = pl.kernel(
        _dot_body,
        out_type=jax.ShapeDtypeStruct((NW * 128,), jnp.float32),
        mesh=mesh,
        scratch_types=[
            pltpu.VMEM((BPW,), jnp.int32),
            pltpu.VMEM((BPW,), jnp.int32),
            pltpu.VMEM((BPW,), jnp.int32),
            pltpu.VMEM((BPW,), jnp.int32),
            pltpu.VMEM((CHUNK, 2 * DIM), jnp.float32),
            pltpu.VMEM((CHUNK, 2 * DIM), jnp.float32),
            pltpu.VMEM((128,), jnp.float32),
            pltpu.SemaphoreType.DMA,
            pltpu.SemaphoreType.DMA,
        ],
    )
    return f(w_i, w_j, wp, cp)


def _bias_body(w_i_hbm, w_j_hbm, w_bias_hbm, c_bias_hbm, bias_hbm,
               idx_i_v, idx_j_v, bi_v, bj_v, sem_bi, sem_bj):
    wid = lax.axis_index("s") * NC + lax.axis_index("c")
    base = wid * BPW

    pltpu.sync_copy(w_i_hbm.at[pl.ds(base, BPW)], idx_i_v)
    pltpu.sync_copy(w_j_hbm.at[pl.ds(base, BPW)], idx_j_v)

    cp_bi = pltpu.async_copy(w_bias_hbm.at[idx_i_v], bi_v, sem_bi)
    cp_bj = pltpu.async_copy(c_bias_hbm.at[idx_j_v], bj_v, sem_bj)
    cp_bi.wait()
    cp_bj.wait()

    def bias_step(k, _):
        s = pl.ds(k * L, L)
        bi_v[s] = bi_v[s] + bj_v[s]
        return 0

    lax.fori_loop(0, BPW // L, bias_step, 0, unroll=4)
    pltpu.sync_copy(bi_v, bias_hbm.at[pl.ds(base, BPW)])


def _sc_bias(w_i, w_j, w_bias, c_bias):
    mesh = plsc.VectorSubcoreMesh(core_axis_name="c", subcore_axis_name="s")
    f = pl.kernel(
        _bias_body,
        out_type=jax.ShapeDtypeStruct((BATCH,), jnp.float32),
        mesh=mesh,
        compiler_params=pltpu.CompilerParams(use_tc_tiling_on_sc=False),
        scratch_types=[
            pltpu.VMEM((BPW,), jnp.int32),
            pltpu.VMEM((BPW,), jnp.int32),
            pltpu.VMEM((BPW,), jnp.float32),
            pltpu.VMEM((BPW,), jnp.float32),
            pltpu.SemaphoreType.DMA,
            pltpu.SemaphoreType.DMA,
        ],
    )
    return f(w_i, w_j, w_bias, c_bias)


_PACK_W = 4096
_PACK_H = _PACK_W // 2
_PACK_GRID = (VOCAB + _PACK_W - 1) // _PACK_W
_PACK_ROWS = _PACK_GRID * _PACK_H


def _transpose_mxu(x):
    eye = (lax.broadcasted_iota(jnp.int32, (DIM, DIM), 0)
           == lax.broadcasted_iota(jnp.int32, (DIM, DIM), 1)
           ).astype(jnp.float32)
    return lax.dot_general(x, eye, (((0,), (0,)), ((), ())),
                           preferred_element_type=jnp.float32)


def _pack_body(wt_ref, ct_ref, ow_ref, oc_ref):
    w = wt_ref[...]
    ow_ref[...] = jnp.concatenate(
        [_transpose_mxu(w[:, :_PACK_H]), _transpose_mxu(w[:, _PACK_H:])],
        axis=1)
    c = ct_ref[...]
    oc_ref[...] = jnp.concatenate(
        [_transpose_mxu(c[:, :_PACK_H]), _transpose_mxu(c[:, _PACK_H:])],
        axis=1)


def _tc_pack(wt, ct):
    return pl.pallas_call(
        _pack_body,
        grid=(_PACK_GRID,),
        in_specs=[
            pl.BlockSpec((DIM, _PACK_W), lambda g: (0, g)),
            pl.BlockSpec((DIM, _PACK_W), lambda g: (0, g)),
        ],
        out_specs=[
            pl.BlockSpec((_PACK_H, 128), lambda g: (g, 0)),
            pl.BlockSpec((_PACK_H, 128), lambda g: (g, 0)),
        ],
        out_shape=(
            jax.ShapeDtypeStruct((_PACK_ROWS, 128), jnp.float32),
            jax.ShapeDtypeStruct((_PACK_ROWS, 128), jnp.float32),
        ),
    )(wt, ct)


def _tc_body(partials_ref, bias_ref, x_ref, loss_ref):
    x = jnp.sum(partials_ref[...])
    b = bias_ref[...]
    y_true = jnp.abs(b) + 1e-6
    t = jnp.abs(x) / 100.0 + jnp.zeros_like(b)
    weight = jnp.exp(0.75 * jnp.log(t))
    loss_ref[...] = weight * jnp.square(x - jnp.log(y_true))
    x_ref[...] = jnp.broadcast_to(x, (1, 1))


def _tc_loss(partials, bias2d):
    return pl.pallas_call(
        _tc_body,
        out_shape=(
            jax.ShapeDtypeStruct((1, 1), jnp.float32),
            jax.ShapeDtypeStruct(bias2d.shape, jnp.float32),
        ),
    )(partials, bias2d)


def kernel(w_i, w_j, w_emb, c_emb, w_bias, c_bias):
    w_i = w_i.astype(jnp.int32)
    w_j = w_j.astype(jnp.int32)
    wp, cp = _tc_pack(w_emb.T, c_emb.T)
    partials = _sc_dot(w_i, w_j, wp, cp)
    bias = _sc_bias(w_i, w_j, w_bias, c_bias)
    x, loss = _tc_loss(partials.reshape(NW, 128), bias.reshape(128, 128))
    return (x.reshape(()), loss.reshape(BATCH))


# speedup vs baseline: 2.0687x; 1.0025x over previous
import jax
import jax.numpy as jnp
from jax import lax
from jax.experimental import pallas as pl
from jax.experimental.pallas import tpu as pltpu
from jax.experimental.pallas import tpu_sc as plsc

VOCAB = 1000000
DIM = 64
BATCH = 16384

_info = plsc.get_sparse_core_info()
NC, NS, L = _info.num_cores, _info.num_subcores, _info.num_lanes
NW = NC * NS
BPW = BATCH // NW
CHUNK = 128


def _dot_body(w_i_hbm, w_j_hbm, wp_hbm, cp_hbm, partials_hbm,
              idx_i_v, idx_j_v, row_i_v, row_j_v, rows_i_v, rows_j_v,
              acc_v, sem_i, sem_j):
    wid = lax.axis_index("s") * NC + lax.axis_index("c")
    base = wid * BPW

    pltpu.sync_copy(w_i_hbm.at[pl.ds(base, BPW)], idx_i_v)
    pltpu.sync_copy(w_j_hbm.at[pl.ds(base, BPW)], idx_j_v)

    def to_rows(k, _):
        s = pl.ds(k * L, L)
        iv = idx_i_v[s]
        jv = idx_j_v[s]
        row_i_v[s] = ((iv >> 12) << 11) | (iv & 2047)
        row_j_v[s] = ((jv >> 12) << 11) | (jv & 2047)
        return 0

    lax.fori_loop(0, BPW // L, to_rows, 0, unroll=4)

    zero = jnp.zeros((L,), jnp.float32)
    accs = (zero, zero, zero, zero)
    for chunk in range(BPW // CHUNK):
        cb = chunk * CHUNK
        cp_i = pltpu.async_copy(
            wp_hbm.at[row_i_v.at[pl.ds(cb, CHUNK)]], rows_i_v, sem_i)
        cp_j = pltpu.async_copy(
            cp_hbm.at[row_j_v.at[pl.ds(cb, CHUNK)]], rows_j_v, sem_j)
        cp_i.wait()
        cp_j.wait()

        def dot_group(g, accs):
            a0, a1, a2, a3 = accs
            iv = idx_i_v[pl.ds(cb + g * L, L)]
            jv = idx_j_v[pl.ds(cb + g * L, L)]
            for t in range(L):
                k = g * L + t
                oi = ((iv[t] >> 11) & 1) * DIM
                oj = ((jv[t] >> 11) & 1) * DIM
                a0 = a0 + rows_i_v[k, pl.ds(oi, L)] * rows_j_v[k, pl.ds(oj, L)]
                a1 = a1 + (rows_i_v[k, pl.ds(oi + L, L)]
                           * rows_j_v[k, pl.ds(oj + L, L)])
                a2 = a2 + (rows_i_v[k, pl.ds(oi + 2 * L, L)]
                           * rows_j_v[k, pl.ds(oj + 2 * L, L)])
                a3 = a3 + (rows_i_v[k, pl.ds(oi + 3 * L, L)]
                           * rows_j_v[k, pl.ds(oj + 3 * L, L)])
            return (a0, a1, a2, a3)

        accs = lax.fori_loop(0, CHUNK // L, dot_group, accs)

    a0, a1, a2, a3 = accs
    acc_v[pl.ds(0, L)] = a0
    acc_v[pl.ds(L, L)] = a1
    acc_v[pl.ds(2 * L, L)] = a2
    acc_v[pl.ds(3 * L, L)] = a3
    for z in range(4, 8):
        acc_v[pl.ds(z * L, L)] = zero
    pltpu.sync_copy(acc_v, partials_hbm.at[pl.ds(wid * 128, 128)])


def _sc_dot(w_i, w_j, wp, cp):
    mesh = plsc.VectorSubcoreMesh(core_axis_name="c", subcore_axis_name="s")
    f = pl.kernel(
        _dot_body,
        out_type=jax.ShapeDtypeStruct((NW * 128,), jnp.float32),
        mesh=mesh,
        scratch_types=[
            pltpu.VMEM((BPW,), jnp.int32),
            pltpu.VMEM((BPW,), jnp.int32),
            pltpu.VMEM((BPW,), jnp.int32),
            pltpu.VMEM((BPW,), jnp.int32),
            pltpu.VMEM((CHUNK, 2 * DIM), jnp.float32),
            pltpu.VMEM((CHUNK, 2 * DIM), jnp.float32),
            pltpu.VMEM((128,), jnp.float32),
            pltpu.SemaphoreType.DMA,
            pltpu.SemaphoreType.DMA,
        ],
    )
    return f(w_i, w_j, wp, cp)


def _bias_body(w_i_hbm, w_j_hbm, w_bias_hbm, c_bias_hbm, bias_hbm,
               idx_i_v, idx_j_v, bi_v, bj_v, sem_bi, sem_bj):
    wid = lax.axis_index("s") * NC + lax.axis_index("c")
    base = wid * BPW

    pltpu.sync_copy(w_i_hbm.at[pl.ds(base, BPW)], idx_i_v)
    pltpu.sync_copy(w_j_hbm.at[pl.ds(base, BPW)], idx_j_v)

    cp_bi = pltpu.async_copy(w_bias_hbm.at[idx_i_v], bi_v, sem_bi)
    cp_bj = pltpu.async_copy(c_bias_hbm.at[idx_j_v], bj_v, sem_bj)
    cp_bi.wait()
    cp_bj.wait()

    def bias_step(k, _):
        s = pl.ds(k * L, L)
        bi_v[s] = bi_v[s] + bj_v[s]
        return 0

    lax.fori_loop(0, BPW // L, bias_step, 0, unroll=4)
    pltpu.sync_copy(bi_v, bias_hbm.at[pl.ds(base, BPW)])


def _sc_bias(w_i, w_j, w_bias, c_bias):
    mesh = plsc.VectorSubcoreMesh(core_axis_name="c", subcore_axis_name="s")
    f = pl.kernel(
        _bias_body,
        out_type=jax.ShapeDtypeStruct((BATCH,), jnp.float32),
        mesh=mesh,
        compiler_params=pltpu.CompilerParams(use_tc_tiling_on_sc=False),
        scratch_types=[
            pltpu.VMEM((BPW,), jnp.int32),
            pltpu.VMEM((BPW,), jnp.int32),
            pltpu.VMEM((BPW,), jnp.float32),
            pltpu.VMEM((BPW,), jnp.float32),
            pltpu.SemaphoreType.DMA,
            pltpu.SemaphoreType.DMA,
        ],
    )
    return f(w_i, w_j, w_bias, c_bias)


_PACK_W = 4096
_PACK_H = _PACK_W // 2
_PACK_GRID = (VOCAB + _PACK_W - 1) // _PACK_W
_PACK_ROWS = _PACK_GRID * _PACK_H


def _pack_body(wt_ref, ct_ref, ow_ref, oc_ref):
    w = wt_ref[...]
    ow_ref[...] = jnp.concatenate(
        [w[:, :_PACK_H].T, w[:, _PACK_H:].T], axis=1)
    c = ct_ref[...]
    oc_ref[...] = jnp.concatenate(
        [c[:, :_PACK_H].T, c[:, _PACK_H:].T], axis=1)


def _tc_pack(wt, ct):
    return pl.pallas_call(
        _pack_body,
        grid=(_PACK_GRID,),
        in_specs=[
            pl.BlockSpec((DIM, _PACK_W), lambda g: (0, g)),
            pl.BlockSpec((DIM, _PACK_W), lambda g: (0, g)),
        ],
        out_specs=[
            pl.BlockSpec((_PACK_H, 128), lambda g: (g, 0)),
            pl.BlockSpec((_PACK_H, 128), lambda g: (g, 0)),
        ],
        out_shape=(
            jax.ShapeDtypeStruct((_PACK_ROWS, 128), jnp.float32),
            jax.ShapeDtypeStruct((_PACK_ROWS, 128), jnp.float32),
        ),
    )(wt, ct)


def _tc_body(partials_ref, bias_ref, x_ref, loss_ref):
    x = jnp.sum(partials_ref[...])
    b = bias_ref[...]
    y_true = jnp.abs(b) + 1e-6
    t = jnp.abs(x) / 100.0 + jnp.zeros_like(b)
    weight = jnp.exp(0.75 * jnp.log(t))
    loss_ref[...] = weight * jnp.square(x - jnp.log(y_true))
    x_ref[...] = jnp.broadcast_to(x, (1, 1))


def _tc_loss(partials, bias2d):
    return pl.pallas_call(
        _tc_body,
        out_shape=(
            jax.ShapeDtypeStruct((1, 1), jnp.float32),
            jax.ShapeDtypeStruct(bias2d.shape, jnp.float32),
        ),
    )(partials, bias2d)


def kernel(w_i, w_j, w_emb, c_emb, w_bias, c_bias):
    w_i = w_i.astype(jnp.int32)
    w_j = w_j.astype(jnp.int32)
    wp, cp = _tc_pack(w_emb.T, c_emb.T)
    partials = _sc_dot(w_i, w_j, wp, cp)
    bias = _sc_bias(w_i, w_j, w_bias, c_bias)
    x, loss = _tc_loss(partials.reshape(NW, 128), bias.reshape(128, 128))
    return (x.reshape(()), loss.reshape(BATCH))
